# baseline (speedup 1.0000x reference)
import jax
import jax.numpy as jnp
from jax import lax
from jax.experimental import pallas as pl
from jax.experimental.pallas import tpu as pltpu
from jax.experimental.pallas import tpu_sc as plsc

NC = 2
NS = 16
LANES = 16
C = 128
NBUF = 3


def _sc_segment_sum(n_feat, src_r, dst_r, w_r, n_chunks):
    N, D = n_feat.shape
    Dh = D // 2
    rpt = -(-N // (NS * 8)) * 8
    Npad = rpt * NS
    n_full = rpt // C
    tail = rpt - n_full * C
    mesh = plsc.VectorSubcoreMesh(
        core_axis_name="c", subcore_axis_name="s",
        num_cores=NC, num_subcores=NS)

    def body(nfeat_hbm, src_hbm, dst_hbm, w_hbm, out_hbm,
             src_v, dst_v, w_v, rows_0, rows_1, rows_2, neigh_sh,
             gsem, ssem):
        cid = lax.axis_index("c")
        sid = lax.axis_index("s")
        bufs = (rows_0, rows_1, rows_2)

        pltpu.sync_copy(src_hbm.at[sid], src_v)
        pltpu.sync_copy(dst_hbm.at[sid], dst_v)
        pltpu.sync_copy(w_hbm.at[sid], w_v)

        def trow(r, _):
            for c8 in range(C // LANES):
                sl = pl.ds(c8 * LANES, LANES)
                v = src_v[r, sl]
                src_v[r, sl] = v + v + cid
            return 0

        lax.fori_loop(0, n_chunks + 1, trow, 0)

        zeros = jnp.zeros((LANES,), jnp.float32)

        def zrow(i, _):
            for k in range(Dh // LANES):
                rows_0[i, pl.ds(k * LANES, LANES)] = zeros
            return 0

        lax.fori_loop(0, C, zrow, 0)
        base = sid * rpt
        for k in range(n_full):
            pltpu.sync_copy(rows_0, neigh_sh.at[pl.ds(base + k * C, C)])
        if tail:
            pltpu.sync_copy(rows_0.at[pl.ds(0, tail)],
                            neigh_sh.at[pl.ds(base + n_full * C, tail)])
        plsc.subcore_barrier()

        def scale(buf, j):
            def group(g, _):
                wv = w_v[j, pl.ds(g * LANES, LANES)]
                for l in range(LANES):
                    ws = wv[l]
                    i = g * LANES + l
                    for k in range(Dh // LANES):
                        sl = pl.ds(k * LANES, LANES)
                        buf[i, sl] = buf[i, sl] * ws
                return 0

            lax.fori_loop(0, C // LANES, group, 0)

        pltpu.async_copy(nfeat_hbm.at[src_v.at[0]], rows_0, gsem)

        def step(t, _):
            for b in range(NBUF):
                j = t * NBUF + b
                cur = bufs[b]
                nxt = bufs[(b + 1) % NBUF]

                @pl.when(j >= 2)
                def _():
                    pltpu.make_async_copy(
                        nxt, neigh_sh.at[dst_v.at[0]], ssem).wait()

                pltpu.async_copy(nfeat_hbm.at[src_v.at[j + 1]], nxt, gsem)
                pltpu.make_async_copy(
                    nfeat_hbm.at[src_v.at[j]], cur, gsem).wait()
                scale(cur, j)
                pltpu.async_copy(cur, neigh_sh.at[dst_v.at[j]], ssem,
                                 add=True)
            return 0

        lax.fori_loop(0, n_chunks // NBUF, step, 0)
        pltpu.make_async_copy(
            nfeat_hbm.at[src_v.at[n_chunks]], rows_0, gsem).wait()
        pltpu.make_async_copy(rows_0, neigh_sh.at[dst_v.at[0]], ssem).wait()
        pltpu.make_async_copy(rows_0, neigh_sh.at[dst_v.at[0]], ssem).wait()

        plsc.subcore_barrier()
        pltpu.sync_copy(neigh_sh.at[pl.ds(base, rpt)],
                        out_hbm.at[cid, pl.ds(base, rpt)])

    run = pl.kernel(
        body,
        out_type=jax.ShapeDtypeStruct((NC, Npad, Dh), jnp.float32),
        mesh=mesh,
        compiler_params=pltpu.CompilerParams(use_tc_tiling_on_sc=False),
        scratch_types=[
            pltpu.VMEM((n_chunks + 1, C), jnp.int32),
            pltpu.VMEM((n_chunks, C), jnp.int32),
            pltpu.VMEM((n_chunks, C), jnp.float32),
            pltpu.VMEM((C, Dh), jnp.float32),
            pltpu.VMEM((C, Dh), jnp.float32),
            pltpu.VMEM((C, Dh), jnp.float32),
            pltpu.VMEM_SHARED((Npad, Dh), jnp.float32),
            pltpu.SemaphoreType.DMA,
            pltpu.SemaphoreType.DMA,
        ],
    )
    return run(n_feat.reshape(2 * N, Dh), src_r, dst_r, w_r)


def _tc_mlp(n_feat, partials, W1, b1, W2, b2):
    N, D = n_feat.shape
    Dh = D // 2
    BLK = 400
    grid = N // BLK

    def body(nf_ref, pp_ref, w1_ref, b1_ref, w2_ref, b2_ref, out_ref):
        neigh = jnp.concatenate([pp_ref[0], pp_ref[1]], axis=-1)
        rst = nf_ref[...] + neigh
        h = jnp.dot(rst, w1_ref[...], preferred_element_type=jnp.float32)
        h = jnp.maximum(h + b1_ref[...], 0.0)
        o = jnp.dot(h, w2_ref[...], preferred_element_type=jnp.float32)
        out_ref[...] = o + b2_ref[...]

    return pl.pallas_call(
        body,
        grid=(grid,),
        in_specs=[
            pl.BlockSpec((BLK, D), lambda i: (i, 0)),
            pl.BlockSpec((NC, BLK, Dh), lambda i: (0, i, 0)),
            pl.BlockSpec((D, D), lambda i: (0, 0)),
            pl.BlockSpec((1, D), lambda i: (0, 0)),
            pl.BlockSpec((D, D), lambda i: (0, 0)),
            pl.BlockSpec((1, D), lambda i: (0, 0)),
        ],
        out_specs=pl.BlockSpec((BLK, D), lambda i: (i, 0)),
        out_shape=jax.ShapeDtypeStruct((N, D), jnp.float32),
    )(n_feat, partials, W1, b1.reshape(1, D), W2, b2.reshape(1, D))


@jax.jit
def kernel(n_feat, e_feat, edge_weight, edge_index, W1, b1, W2, b2):
    del e_feat
    N, D = n_feat.shape
    E = edge_index.shape[1]
    epw = -(-E // NS)
    epw = -(-epw // (NBUF * C)) * (NBUF * C)
    E_pad = epw * NS
    pad = E_pad - E

    src = edge_index[0].astype(jnp.int32)
    dst = edge_index[1].astype(jnp.int32)
    w = edge_weight[:, 0].astype(jnp.float32)
    if pad:
        src = jnp.concatenate([src, jnp.zeros((pad,), jnp.int32)])
        dst = jnp.concatenate([dst, jnp.zeros((pad,), jnp.int32)])
        w = jnp.concatenate([w, jnp.zeros((pad,), jnp.float32)])

    n_chunks = epw // C
    src_r = src.reshape(NS, n_chunks, C)
    src_r = jnp.concatenate(
        [src_r, jnp.zeros((NS, 1, C), jnp.int32)], axis=1)
    dst_r = dst.reshape(NS, n_chunks, C)
    w_r = w.reshape(NS, n_chunks, C)

    partials = _sc_segment_sum(n_feat, src_r, dst_r, w_r, n_chunks)
    return _tc_mlp(n_feat, partials, W1, b1, W2, b2)
